# Initial kernel scaffold; baseline (speedup 1.0000x reference)
import jax
import jax.numpy as jnp
from jax.experimental import pallas as pl


def kernel(x, edge_index, emb, W0, b0, W1, b1, W2, b2, Wout, bout):
    raise NotImplementedError("write your pallas kernel here")


# speedup vs baseline: 15.0982x; 15.0982x over previous
import functools

import jax
import jax.numpy as jnp
from jax import lax
from jax.experimental import pallas as pl
from jax.experimental.pallas import tpu as pltpu
from jax.experimental.pallas import tpu_sc as plsc

N = 100000
E = 1600000
VOCAB = 128
D = 30
C = 5
DP = 32
H = 16

NC, NS, L = 2, 16, 16
NW = NC * NS

NPAD = 100352
RSC = NPAD // NS

CH = 8
R = 12544
EPAD = R * 128
RT = R // NW
NIT = RT // CH

BLK = 3136
GRID = NPAD // BLK


def _sc_mesh():
    return plsc.VectorSubcoreMesh(
        core_axis_name="c", subcore_axis_name="s",
        num_cores=NC, num_subcores=NS)


def _deg_body(srcd, zeros1, degp, idx_v, ones_v, accum):
    cid = lax.axis_index("c")
    sid = lax.axis_index("s")
    wid = sid * NC + cid
    for i in range(128 // L):
        ones_v[pl.ds(i * L, L)] = jnp.ones((L,), jnp.float32)
    r0 = sid * RSC
    pltpu.sync_copy(zeros1.at[pl.ds(r0, RSC)], accum.at[pl.ds(r0, RSC)])
    plsc.subcore_barrier()
    base = wid * RT

    def it(t, carry):
        row = base + t * CH
        pltpu.sync_copy(srcd.at[pl.ds(row, CH)], idx_v)
        for j in range(CH):
            pltpu.sync_copy(ones_v, accum.at[idx_v.at[j]], add=True)
        return carry

    lax.fori_loop(0, NIT, it, 0)
    plsc.subcore_barrier()
    pltpu.sync_copy(accum.at[pl.ds(r0, RSC)],
                    degp.at[pl.ds(cid * NPAD + r0, RSC)])


_deg_call = pl.kernel(
    _deg_body,
    out_type=jax.ShapeDtypeStruct((NC * NPAD,), jnp.float32),
    mesh=_sc_mesh(),
    scratch_types=[
        pltpu.VMEM((CH, 128), jnp.int32),
        pltpu.VMEM((128,), jnp.float32),
        pltpu.VMEM_SHARED((NPAD,), jnp.float32),
    ],
    compiler_params=pltpu.CompilerParams(use_tc_tiling_on_sc=False),
)


def _agg_body(hnA, hnB, srcg, dsts, zeros2, pA, pB,
              sidx, didx, rows, accum, semg, sems):
    cid = lax.axis_index("c")
    sid = lax.axis_index("s")
    wid = sid * NC + cid
    r0 = sid * RSC
    base = wid * RT
    for half in range(2):
        hn = (hnA, hnB)[half]
        out = (pA, pB)[half]

        @pl.when(cid == 0)
        def _():
            pltpu.sync_copy(hn.at[pl.ds(r0, RSC)], accum.at[pl.ds(r0, RSC)])

        @pl.when(cid != 0)
        def _():
            pltpu.sync_copy(zeros2.at[pl.ds(r0, RSC)], accum.at[pl.ds(r0, RSC)])

        plsc.subcore_barrier()

        def it(t, carry):
            row = base + t * CH
            pltpu.sync_copy(srcg.at[pl.ds(row, CH)], sidx)
            pltpu.sync_copy(dsts.at[pl.ds(row, CH)], didx)
            ds_g = [pltpu.async_copy(hn.at[sidx.at[j]], rows.at[j], semg)
                    for j in range(CH)]
            for d in ds_g:
                d.wait()
            ds_s = [pltpu.async_copy(rows.at[j], accum.at[didx.at[j]], sems,
                                     add=True)
                    for j in range(CH)]
            for d in ds_s:
                d.wait()
            return carry

        lax.fori_loop(0, NIT, it, 0)
        plsc.subcore_barrier()
        pltpu.sync_copy(accum.at[pl.ds(r0, RSC)], out.at[cid, pl.ds(r0, RSC)])
        if half == 0:
            plsc.subcore_barrier()


_agg_call = pl.kernel(
    _agg_body,
    out_type=(jax.ShapeDtypeStruct((NC, NPAD, H), jnp.float32),
              jax.ShapeDtypeStruct((NC, NPAD, H), jnp.float32)),
    mesh=_sc_mesh(),
    scratch_types=[
        pltpu.VMEM((CH, 128), jnp.int32),
        pltpu.VMEM((CH, 128), jnp.int32),
        pltpu.VMEM((CH, 128, H), jnp.float32),
        pltpu.VMEM_SHARED((NPAD, H), jnp.float32),
        pltpu.SemaphoreType.DMA,
        pltpu.SemaphoreType.DMA,
    ],
    compiler_params=pltpu.CompilerParams(use_tc_tiling_on_sc=False),
)


def _init_body(x_ref, emb_ref, d_ref, hnA_ref, hnB_ref, inv_ref):
    i = pl.program_id(0)
    xb = x_ref[...]
    onehot = (xb == lax.broadcasted_iota(jnp.int32, (BLK, VOCAB), 1)
              ).astype(jnp.float32)
    h0 = jnp.dot(onehot, emb_ref[...], preferred_element_type=jnp.float32)
    deg = d_ref[:, 0:1] + d_ref[:, 1:2] + 1.0
    row = i * BLK + lax.broadcasted_iota(jnp.int32, (BLK, 1), 0)
    inv = jnp.where(row < N, 1.0 / deg, 0.0)
    hn = h0 * inv
    hnA_ref[...] = hn[:, :H]
    hnB_ref[...] = hn[:, H:]
    inv_ref[...] = inv


_init_call = pl.pallas_call(
    _init_body,
    grid=(GRID,),
    in_specs=[
        pl.BlockSpec((BLK, 1), lambda i: (i, 0)),
        pl.BlockSpec((VOCAB, DP), lambda i: (0, 0)),
        pl.BlockSpec((BLK, NC), lambda i: (i, 0)),
    ],
    out_specs=[
        pl.BlockSpec((BLK, H), lambda i: (i, 0)),
        pl.BlockSpec((BLK, H), lambda i: (i, 0)),
        pl.BlockSpec((BLK, 1), lambda i: (i, 0)),
    ],
    out_shape=[
        jax.ShapeDtypeStruct((NPAD, H), jnp.float32),
        jax.ShapeDtypeStruct((NPAD, H), jnp.float32),
        jax.ShapeDtypeStruct((NPAD, 1), jnp.float32),
    ],
)


def _mid_body(pA_ref, pB_ref, inv_ref, w_ref, b_ref, hnA_ref, hnB_ref):
    aggr = jnp.concatenate([pA_ref[0] + pA_ref[1], pB_ref[0] + pB_ref[1]],
                           axis=1)
    h = jnp.maximum(
        jnp.dot(aggr, w_ref[...], preferred_element_type=jnp.float32)
        + b_ref[...], 0.0)
    hn = h * inv_ref[...]
    hnA_ref[...] = hn[:, :H]
    hnB_ref[...] = hn[:, H:]


_mid_call = pl.pallas_call(
    _mid_body,
    grid=(GRID,),
    in_specs=[
        pl.BlockSpec((NC, BLK, H), lambda i: (0, i, 0)),
        pl.BlockSpec((NC, BLK, H), lambda i: (0, i, 0)),
        pl.BlockSpec((BLK, 1), lambda i: (i, 0)),
        pl.BlockSpec((DP, DP), lambda i: (0, 0)),
        pl.BlockSpec((1, DP), lambda i: (0, 0)),
    ],
    out_specs=[
        pl.BlockSpec((BLK, H), lambda i: (i, 0)),
        pl.BlockSpec((BLK, H), lambda i: (i, 0)),
    ],
    out_shape=[
        jax.ShapeDtypeStruct((NPAD, H), jnp.float32),
        jax.ShapeDtypeStruct((NPAD, H), jnp.float32),
    ],
)


def _fin_body(pA_ref, pB_ref, w_ref, b_ref, wo_ref, bo_ref, out_ref):
    aggr = jnp.concatenate([pA_ref[0] + pA_ref[1], pB_ref[0] + pB_ref[1]],
                           axis=1)
    h = jnp.maximum(
        jnp.dot(aggr, w_ref[...], preferred_element_type=jnp.float32)
        + b_ref[...], 0.0)
    out_ref[...] = (jnp.dot(h, wo_ref[...], preferred_element_type=jnp.float32)
                    + bo_ref[...])


_fin_call = pl.pallas_call(
    _fin_body,
    grid=(GRID,),
    in_specs=[
        pl.BlockSpec((NC, BLK, H), lambda i: (0, i, 0)),
        pl.BlockSpec((NC, BLK, H), lambda i: (0, i, 0)),
        pl.BlockSpec((DP, DP), lambda i: (0, 0)),
        pl.BlockSpec((1, DP), lambda i: (0, 0)),
        pl.BlockSpec((DP, 8), lambda i: (0, 0)),
        pl.BlockSpec((1, 8), lambda i: (0, 0)),
    ],
    out_specs=pl.BlockSpec((BLK, 8), lambda i: (i, 0)),
    out_shape=jax.ShapeDtypeStruct((NPAD, 8), jnp.float32),
)


def kernel(x, edge_index, emb, W0, b0, W1, b1, W2, b2, Wout, bout):
    f32 = jnp.float32
    x = x.astype(jnp.int32)
    src = edge_index[0].astype(jnp.int32)
    dst = edge_index[1].astype(jnp.int32)
    pad = EPAD - E
    srcg = jnp.concatenate([src, jnp.zeros((pad,), jnp.int32)]).reshape(R, 128)
    dsts = jnp.concatenate([dst, jnp.full((pad,), N, jnp.int32)]).reshape(R, 128)
    srcd = jnp.concatenate([src, jnp.full((pad,), N, jnp.int32)]).reshape(R, 128)
    zeros1 = jnp.zeros((NPAD,), f32)
    zeros2 = jnp.zeros((NPAD, H), f32)
    xp = jnp.concatenate([x, jnp.zeros((NPAD - N,), jnp.int32)]).reshape(NPAD, 1)
    embp = jnp.zeros((VOCAB, DP), f32).at[:, :D].set(emb.astype(f32))

    def padw(w):
        return jnp.zeros((DP, DP), f32).at[:D, :D].set(w.astype(f32).T)

    def padb(b):
        return jnp.zeros((1, DP), f32).at[0, :D].set(b.astype(f32))

    wo = jnp.zeros((DP, 8), f32).at[:D, :C].set(Wout.astype(f32).T)
    bo = jnp.zeros((1, 8), f32).at[0, :C].set(bout.astype(f32))

    degp = _deg_call(srcd, zeros1)
    degt = degp.reshape(NC, NPAD).T
    hnA, hnB, inv = _init_call(xp, embp, degt)
    for w, b in ((W0, b0), (W1, b1)):
        pA, pB = _agg_call(hnA, hnB, srcg, dsts, zeros2)
        hnA, hnB = _mid_call(pA, pB, inv, padw(w), padb(b))
    pA, pB = _agg_call(hnA, hnB, srcg, dsts, zeros2)
    out = _fin_call(pA, pB, padw(W2), padb(b2), wo, bo)
    return out[:N, :C]


# speedup vs baseline: 15.8216x; 1.0479x over previous
import functools

import jax
import jax.numpy as jnp
from jax import lax
from jax.experimental import pallas as pl
from jax.experimental.pallas import tpu as pltpu
from jax.experimental.pallas import tpu_sc as plsc

N = 100000
E = 1600000
VOCAB = 128
D = 30
C = 5
DP = 32
H = 16

NC, NS, L = 2, 16, 16
NW = NC * NS

NPAD = 100352
RSC = NPAD // NS

CH = 8
R = 12544
EPAD = R * 128
RT = R // NW
NIT = RT // CH

BLK = 3136
GRID = NPAD // BLK


def _sc_mesh():
    return plsc.VectorSubcoreMesh(
        core_axis_name="c", subcore_axis_name="s",
        num_cores=NC, num_subcores=NS)


def _deg_body(srcd, zeros1, degp, idx_v, ones_v, accum):
    cid = lax.axis_index("c")
    sid = lax.axis_index("s")
    wid = sid * NC + cid
    for i in range(128 // L):
        ones_v[pl.ds(i * L, L)] = jnp.ones((L,), jnp.float32)
    r0 = sid * RSC
    pltpu.sync_copy(zeros1.at[pl.ds(r0, RSC)], accum.at[pl.ds(r0, RSC)])
    plsc.subcore_barrier()
    base = wid * RT

    def it(t, carry):
        row = base + t * CH
        pltpu.sync_copy(srcd.at[pl.ds(row, CH)], idx_v)
        for j in range(CH):
            pltpu.sync_copy(ones_v, accum.at[idx_v.at[j]], add=True)
        return carry

    lax.fori_loop(0, NIT, it, 0)
    plsc.subcore_barrier()
    pltpu.sync_copy(accum.at[pl.ds(r0, RSC)],
                    degp.at[pl.ds(cid * NPAD + r0, RSC)])


_deg_call = pl.kernel(
    _deg_body,
    out_type=jax.ShapeDtypeStruct((NC * NPAD,), jnp.float32),
    mesh=_sc_mesh(),
    scratch_types=[
        pltpu.VMEM((CH, 128), jnp.int32),
        pltpu.VMEM((128,), jnp.float32),
        pltpu.VMEM_SHARED((NPAD,), jnp.float32),
    ],
    compiler_params=pltpu.CompilerParams(use_tc_tiling_on_sc=False),
)


CE = 512
NCHUNK = RT * 128 // CE


def _agg_body(hnA, hnB, srcg, dsts, zeros2, pA, pB,
              sA, dA, sB, dB, rowsA, rowsB, accum,
              semgA, semgB, semsA, semsB):
    cid = lax.axis_index("c")
    sid = lax.axis_index("s")
    wid = sid * NC + cid
    r0 = sid * RSC
    base = wid * RT * 128

    for half in range(2):
        hn = (hnA, hnB)[half]
        out = (pA, pB)[half]

        def stage(t, sref, dref, rbuf, semg):
            pltpu.sync_copy(srcg.at[pl.ds(base + t * CE, CE)], sref)
            pltpu.sync_copy(dsts.at[pl.ds(base + t * CE, CE)], dref)
            pltpu.async_copy(hn.at[sref], rbuf, semg)

        def wait_g(sref, rbuf, semg):
            pltpu.make_async_copy(hn.at[sref], rbuf, semg).wait()

        def flush(dref, rbuf, sems):
            pltpu.async_copy(rbuf, accum.at[dref], sems, add=True)

        def wait_s(dref, rbuf, sems):
            pltpu.make_async_copy(rbuf, accum.at[dref], sems).wait()

        @pl.when(cid == 0)
        def _():
            pltpu.sync_copy(hn.at[pl.ds(r0, RSC)], accum.at[pl.ds(r0, RSC)])

        @pl.when(cid != 0)
        def _():
            pltpu.sync_copy(zeros2.at[pl.ds(r0, RSC)], accum.at[pl.ds(r0, RSC)])

        plsc.subcore_barrier()

        stage(0, sA, dA, rowsA, semgA)
        stage(1, sB, dB, rowsB, semgB)

        def it(i, carry):
            t = 2 * i
            wait_g(sA, rowsA, semgA)
            flush(dA, rowsA, semsA)
            wait_g(sB, rowsB, semgB)
            flush(dB, rowsB, semsB)
            wait_s(dA, rowsA, semsA)
            stage(t + 2, sA, dA, rowsA, semgA)
            wait_s(dB, rowsB, semsB)
            stage(t + 3, sB, dB, rowsB, semgB)
            return carry

        lax.fori_loop(0, NCHUNK // 2 - 1, it, 0)
        wait_g(sA, rowsA, semgA)
        flush(dA, rowsA, semsA)
        wait_g(sB, rowsB, semgB)
        flush(dB, rowsB, semsB)
        wait_s(dA, rowsA, semsA)
        wait_s(dB, rowsB, semsB)

        plsc.subcore_barrier()
        pltpu.sync_copy(accum.at[pl.ds(r0, RSC)], out.at[cid, pl.ds(r0, RSC)])
        if half == 0:
            plsc.subcore_barrier()


_agg_call = pl.kernel(
    _agg_body,
    out_type=(jax.ShapeDtypeStruct((NC, NPAD, H), jnp.float32),
              jax.ShapeDtypeStruct((NC, NPAD, H), jnp.float32)),
    mesh=_sc_mesh(),
    scratch_types=[
        pltpu.VMEM((CE,), jnp.int32),
        pltpu.VMEM((CE,), jnp.int32),
        pltpu.VMEM((CE,), jnp.int32),
        pltpu.VMEM((CE,), jnp.int32),
        pltpu.VMEM((CE, H), jnp.float32),
        pltpu.VMEM((CE, H), jnp.float32),
        pltpu.VMEM_SHARED((NPAD, H), jnp.float32),
        pltpu.SemaphoreType.DMA,
        pltpu.SemaphoreType.DMA,
        pltpu.SemaphoreType.DMA,
        pltpu.SemaphoreType.DMA,
    ],
    compiler_params=pltpu.CompilerParams(use_tc_tiling_on_sc=False),
)


def _init_body(x_ref, emb_ref, d_ref, hnA_ref, hnB_ref, inv_ref):
    i = pl.program_id(0)
    xb = x_ref[...]
    onehot = (xb == lax.broadcasted_iota(jnp.int32, (BLK, VOCAB), 1)
              ).astype(jnp.float32)
    h0 = jnp.dot(onehot, emb_ref[...], preferred_element_type=jnp.float32)
    deg = d_ref[:, 0:1] + d_ref[:, 1:2] + 1.0
    row = i * BLK + lax.broadcasted_iota(jnp.int32, (BLK, 1), 0)
    inv = jnp.where(row < N, 1.0 / deg, 0.0)
    hn = h0 * inv
    hnA_ref[...] = hn[:, :H]
    hnB_ref[...] = hn[:, H:]
    inv_ref[...] = inv


_init_call = pl.pallas_call(
    _init_body,
    grid=(GRID,),
    in_specs=[
        pl.BlockSpec((BLK, 1), lambda i: (i, 0)),
        pl.BlockSpec((VOCAB, DP), lambda i: (0, 0)),
        pl.BlockSpec((BLK, NC), lambda i: (i, 0)),
    ],
    out_specs=[
        pl.BlockSpec((BLK, H), lambda i: (i, 0)),
        pl.BlockSpec((BLK, H), lambda i: (i, 0)),
        pl.BlockSpec((BLK, 1), lambda i: (i, 0)),
    ],
    out_shape=[
        jax.ShapeDtypeStruct((NPAD, H), jnp.float32),
        jax.ShapeDtypeStruct((NPAD, H), jnp.float32),
        jax.ShapeDtypeStruct((NPAD, 1), jnp.float32),
    ],
)


def _mid_body(pA_ref, pB_ref, inv_ref, w_ref, b_ref, hnA_ref, hnB_ref):
    aggr = jnp.concatenate([pA_ref[0] + pA_ref[1], pB_ref[0] + pB_ref[1]],
                           axis=1)
    h = jnp.maximum(
        jnp.dot(aggr, w_ref[...], preferred_element_type=jnp.float32)
        + b_ref[...], 0.0)
    hn = h * inv_ref[...]
    hnA_ref[...] = hn[:, :H]
    hnB_ref[...] = hn[:, H:]


_mid_call = pl.pallas_call(
    _mid_body,
    grid=(GRID,),
    in_specs=[
        pl.BlockSpec((NC, BLK, H), lambda i: (0, i, 0)),
        pl.BlockSpec((NC, BLK, H), lambda i: (0, i, 0)),
        pl.BlockSpec((BLK, 1), lambda i: (i, 0)),
        pl.BlockSpec((DP, DP), lambda i: (0, 0)),
        pl.BlockSpec((1, DP), lambda i: (0, 0)),
    ],
    out_specs=[
        pl.BlockSpec((BLK, H), lambda i: (i, 0)),
        pl.BlockSpec((BLK, H), lambda i: (i, 0)),
    ],
    out_shape=[
        jax.ShapeDtypeStruct((NPAD, H), jnp.float32),
        jax.ShapeDtypeStruct((NPAD, H), jnp.float32),
    ],
)


def _fin_body(pA_ref, pB_ref, w_ref, b_ref, wo_ref, bo_ref, out_ref):
    aggr = jnp.concatenate([pA_ref[0] + pA_ref[1], pB_ref[0] + pB_ref[1]],
                           axis=1)
    h = jnp.maximum(
        jnp.dot(aggr, w_ref[...], preferred_element_type=jnp.float32)
        + b_ref[...], 0.0)
    out_ref[...] = (jnp.dot(h, wo_ref[...], preferred_element_type=jnp.float32)
                    + bo_ref[...])


_fin_call = pl.pallas_call(
    _fin_body,
    grid=(GRID,),
    in_specs=[
        pl.BlockSpec((NC, BLK, H), lambda i: (0, i, 0)),
        pl.BlockSpec((NC, BLK, H), lambda i: (0, i, 0)),
        pl.BlockSpec((DP, DP), lambda i: (0, 0)),
        pl.BlockSpec((1, DP), lambda i: (0, 0)),
        pl.BlockSpec((DP, 8), lambda i: (0, 0)),
        pl.BlockSpec((1, 8), lambda i: (0, 0)),
    ],
    out_specs=pl.BlockSpec((BLK, 8), lambda i: (i, 0)),
    out_shape=jax.ShapeDtypeStruct((NPAD, 8), jnp.float32),
)


def kernel(x, edge_index, emb, W0, b0, W1, b1, W2, b2, Wout, bout):
    f32 = jnp.float32
    x = x.astype(jnp.int32)
    src = edge_index[0].astype(jnp.int32)
    dst = edge_index[1].astype(jnp.int32)
    pad = EPAD - E
    srcg = jnp.concatenate([src, jnp.zeros((pad,), jnp.int32)])
    dsts = jnp.concatenate([dst, jnp.full((pad,), N, jnp.int32)])
    srcd = jnp.concatenate([src, jnp.full((pad,), N, jnp.int32)]).reshape(R, 128)
    zeros1 = jnp.zeros((NPAD,), f32)
    zeros2 = jnp.zeros((NPAD, H), f32)
    xp = jnp.concatenate([x, jnp.zeros((NPAD - N,), jnp.int32)]).reshape(NPAD, 1)
    embp = jnp.zeros((VOCAB, DP), f32).at[:, :D].set(emb.astype(f32))

    def padw(w):
        return jnp.zeros((DP, DP), f32).at[:D, :D].set(w.astype(f32).T)

    def padb(b):
        return jnp.zeros((1, DP), f32).at[0, :D].set(b.astype(f32))

    wo = jnp.zeros((DP, 8), f32).at[:D, :C].set(Wout.astype(f32).T)
    bo = jnp.zeros((1, 8), f32).at[0, :C].set(bout.astype(f32))

    degp = _deg_call(srcd, zeros1)
    degt = degp.reshape(NC, NPAD).T
    hnA, hnB, inv = _init_call(xp, embp, degt)
    for w, b in ((W0, b0), (W1, b1)):
        pA, pB = _agg_call(hnA, hnB, srcg, dsts, zeros2)
        hnA, hnB = _mid_call(pA, pB, inv, padw(w), padb(b))
    pA, pB = _agg_call(hnA, hnB, srcg, dsts, zeros2)
    out = _fin_call(pA, pB, padw(W2), padb(b2), wo, bo)
    return out[:N, :C]


# speedup vs baseline: 21.4891x; 1.3582x over previous
import functools

import jax
import jax.numpy as jnp
from jax import lax
from jax.experimental import pallas as pl
from jax.experimental.pallas import tpu as pltpu
from jax.experimental.pallas import tpu_sc as plsc

N = 100000
E = 1600000
VOCAB = 128
D = 30
C = 5
DP = 32
H = 16

NC, NS, L = 2, 16, 16
NW = NC * NS

NPAD = 100352
RSC = NPAD // NS

CH = 8
R = 12544
EPAD = R * 128
RT = R // NW
NIT = RT // CH

BLK = 2048
GRID = NPAD // BLK


def _sc_mesh():
    return plsc.VectorSubcoreMesh(
        core_axis_name="c", subcore_axis_name="s",
        num_cores=NC, num_subcores=NS)


def _deg_body(srcd, zeros1, degp, idx_v, ones_v, accum):
    cid = lax.axis_index("c")
    sid = lax.axis_index("s")
    wid = sid * NC + cid
    for i in range(128 // L):
        ones_v[pl.ds(i * L, L)] = jnp.ones((L,), jnp.float32)
    r0 = sid * RSC
    pltpu.sync_copy(zeros1.at[pl.ds(r0, RSC)], accum.at[pl.ds(r0, RSC)])
    plsc.subcore_barrier()
    base = wid * RT

    def it(t, carry):
        row = base + t * CH
        pltpu.sync_copy(srcd.at[pl.ds(row, CH)], idx_v)
        for j in range(CH):
            pltpu.sync_copy(ones_v, accum.at[idx_v.at[j]], add=True)
        return carry

    lax.fori_loop(0, NIT, it, 0)
    plsc.subcore_barrier()
    pltpu.sync_copy(accum.at[pl.ds(r0, RSC)],
                    degp.at[pl.ds(cid * NPAD + r0, RSC)])


_deg_call = pl.kernel(
    _deg_body,
    out_type=jax.ShapeDtypeStruct((NC * NPAD,), jnp.float32),
    mesh=_sc_mesh(),
    scratch_types=[
        pltpu.VMEM((CH, 128), jnp.int32),
        pltpu.VMEM((128,), jnp.float32),
        pltpu.VMEM_SHARED((NPAD,), jnp.float32),
    ],
    compiler_params=pltpu.CompilerParams(use_tc_tiling_on_sc=False),
)


CE = 512
NCHUNK = RT * 128 // CE


def _agg_body(hnA, hnB, srcg, dsts, zeros2, pA, pB,
              sA, dA, sB, dB, rowsA, rowsB, accum,
              semgA, semgB, semsA, semsB):
    cid = lax.axis_index("c")
    sid = lax.axis_index("s")
    wid = sid * NC + cid
    r0 = sid * RSC
    base = wid * RT * 128

    for half in range(2):
        hn = (hnA, hnB)[half]
        out = (pA, pB)[half]

        def stage(t, sref, dref, rbuf, semg):
            pltpu.sync_copy(srcg.at[pl.ds(base + t * CE, CE)], sref)
            pltpu.sync_copy(dsts.at[pl.ds(base + t * CE, CE)], dref)
            pltpu.async_copy(hn.at[sref], rbuf, semg)

        def wait_g(sref, rbuf, semg):
            pltpu.make_async_copy(hn.at[sref], rbuf, semg).wait()

        def flush(dref, rbuf, sems):
            pltpu.async_copy(rbuf, accum.at[dref], sems, add=True)

        def wait_s(dref, rbuf, sems):
            pltpu.make_async_copy(rbuf, accum.at[dref], sems).wait()

        @pl.when(cid == 0)
        def _():
            pltpu.sync_copy(hn.at[pl.ds(r0, RSC)], accum.at[pl.ds(r0, RSC)])

        @pl.when(cid != 0)
        def _():
            pltpu.sync_copy(zeros2.at[pl.ds(r0, RSC)], accum.at[pl.ds(r0, RSC)])

        plsc.subcore_barrier()

        stage(0, sA, dA, rowsA, semgA)
        stage(1, sB, dB, rowsB, semgB)

        def it(i, carry):
            t = 2 * i
            wait_g(sA, rowsA, semgA)
            flush(dA, rowsA, semsA)
            wait_g(sB, rowsB, semgB)
            flush(dB, rowsB, semsB)
            wait_s(dA, rowsA, semsA)
            stage(t + 2, sA, dA, rowsA, semgA)
            wait_s(dB, rowsB, semsB)
            stage(t + 3, sB, dB, rowsB, semgB)
            return carry

        lax.fori_loop(0, NCHUNK // 2 - 1, it, 0)
        wait_g(sA, rowsA, semgA)
        flush(dA, rowsA, semsA)
        wait_g(sB, rowsB, semgB)
        flush(dB, rowsB, semsB)
        wait_s(dA, rowsA, semsA)
        wait_s(dB, rowsB, semsB)

        plsc.subcore_barrier()
        pltpu.sync_copy(accum.at[pl.ds(r0, RSC)], out.at[cid, pl.ds(r0, RSC)])
        if half == 0:
            plsc.subcore_barrier()


_agg_call = pl.kernel(
    _agg_body,
    out_type=(jax.ShapeDtypeStruct((NC, NPAD, H), jnp.float32),
              jax.ShapeDtypeStruct((NC, NPAD, H), jnp.float32)),
    mesh=_sc_mesh(),
    scratch_types=[
        pltpu.VMEM((CE,), jnp.int32),
        pltpu.VMEM((CE,), jnp.int32),
        pltpu.VMEM((CE,), jnp.int32),
        pltpu.VMEM((CE,), jnp.int32),
        pltpu.VMEM((CE, H), jnp.float32),
        pltpu.VMEM((CE, H), jnp.float32),
        pltpu.VMEM_SHARED((NPAD, H), jnp.float32),
        pltpu.SemaphoreType.DMA,
        pltpu.SemaphoreType.DMA,
        pltpu.SemaphoreType.DMA,
        pltpu.SemaphoreType.DMA,
    ],
    compiler_params=pltpu.CompilerParams(use_tc_tiling_on_sc=False),
)


BLK8 = BLK // 8
NPAD8 = NPAD // 8


def _init_body(x_ref, emb_ref, d_ref, hnA_ref, hnB_ref, inv_ref):
    i = pl.program_id(0)
    xb = x_ref[...]
    onehot = (xb == lax.broadcasted_iota(jnp.int32, (BLK, VOCAB), 1)
              ).astype(jnp.float32)
    h0 = jnp.dot(onehot, emb_ref[...], preferred_element_type=jnp.float32)
    deg = d_ref[:, 0:1] + d_ref[:, 1:2] + 1.0
    row = i * BLK + lax.broadcasted_iota(jnp.int32, (BLK, 1), 0)
    inv = jnp.where(row < N, 1.0 / deg, 0.0)
    hn = h0 * inv
    hnA_ref[...] = hn[:, :H]
    hnB_ref[...] = hn[:, H:]
    inv_ref[...] = inv


_init_call = pl.pallas_call(
    _init_body,
    grid=(GRID,),
    in_specs=[
        pl.BlockSpec((BLK, 1), lambda i: (i, 0)),
        pl.BlockSpec((VOCAB, DP), lambda i: (0, 0)),
        pl.BlockSpec((BLK, NC), lambda i: (i, 0)),
    ],
    out_specs=[
        pl.BlockSpec((BLK, H), lambda i: (i, 0)),
        pl.BlockSpec((BLK, H), lambda i: (i, 0)),
        pl.BlockSpec((BLK, 1), lambda i: (i, 0)),
    ],
    out_shape=[
        jax.ShapeDtypeStruct((NPAD, H), jnp.float32),
        jax.ShapeDtypeStruct((NPAD, H), jnp.float32),
        jax.ShapeDtypeStruct((NPAD, 1), jnp.float32),
    ],
)


def _hidden_packed(pA_ref, pB_ref, wa_ref, wb_ref, b_ref):
    za = jnp.dot(pA_ref[0] + pA_ref[1], wa_ref[...],
                 preferred_element_type=jnp.float32)
    zb = jnp.dot(pB_ref[0] + pB_ref[1], wb_ref[...],
                 preferred_element_type=jnp.float32)
    return jnp.maximum(za + zb + b_ref[...], 0.0)


def _mid_body(pA_ref, pB_ref, inv_ref, wa_ref, wb_ref, b_ref, sa_ref, sb_ref,
              hnA_ref, hnB_ref):
    h = _hidden_packed(pA_ref, pB_ref, wa_ref, wb_ref, b_ref)
    ip = inv_ref[...]
    hnA_ref[...] = jnp.dot(h, sa_ref[...],
                           preferred_element_type=jnp.float32) * ip
    hnB_ref[...] = jnp.dot(h, sb_ref[...],
                           preferred_element_type=jnp.float32) * ip


_mid_call = pl.pallas_call(
    _mid_body,
    grid=(GRID,),
    in_specs=[
        pl.BlockSpec((NC, BLK8, 128), lambda i: (0, i, 0)),
        pl.BlockSpec((NC, BLK8, 128), lambda i: (0, i, 0)),
        pl.BlockSpec((BLK8, 128), lambda i: (i, 0)),
        pl.BlockSpec((128, 256), lambda i: (0, 0)),
        pl.BlockSpec((128, 256), lambda i: (0, 0)),
        pl.BlockSpec((1, 256), lambda i: (0, 0)),
        pl.BlockSpec((256, 128), lambda i: (0, 0)),
        pl.BlockSpec((256, 128), lambda i: (0, 0)),
    ],
    out_specs=[
        pl.BlockSpec((BLK8, 128), lambda i: (i, 0)),
        pl.BlockSpec((BLK8, 128), lambda i: (i, 0)),
    ],
    out_shape=[
        jax.ShapeDtypeStruct((NPAD8, 128), jnp.float32),
        jax.ShapeDtypeStruct((NPAD8, 128), jnp.float32),
    ],
)


def _fin_body(pA_ref, pB_ref, wa_ref, wb_ref, b_ref, wo_ref, bo_ref, out_ref):
    h = _hidden_packed(pA_ref, pB_ref, wa_ref, wb_ref, b_ref)
    out_ref[...] = (jnp.dot(h, wo_ref[...], preferred_element_type=jnp.float32)
                    + bo_ref[...])


_fin_call = pl.pallas_call(
    _fin_body,
    grid=(GRID,),
    in_specs=[
        pl.BlockSpec((NC, BLK8, 128), lambda i: (0, i, 0)),
        pl.BlockSpec((NC, BLK8, 128), lambda i: (0, i, 0)),
        pl.BlockSpec((128, 256), lambda i: (0, 0)),
        pl.BlockSpec((128, 256), lambda i: (0, 0)),
        pl.BlockSpec((1, 256), lambda i: (0, 0)),
        pl.BlockSpec((256, 64), lambda i: (0, 0)),
        pl.BlockSpec((1, 64), lambda i: (0, 0)),
    ],
    out_specs=pl.BlockSpec((BLK8, 64), lambda i: (i, 0)),
    out_shape=jax.ShapeDtypeStruct((NPAD8, 64), jnp.float32),
)


def kernel(x, edge_index, emb, W0, b0, W1, b1, W2, b2, Wout, bout):
    f32 = jnp.float32
    x = x.astype(jnp.int32)
    src = edge_index[0].astype(jnp.int32)
    dst = edge_index[1].astype(jnp.int32)
    pad = EPAD - E
    srcg = jnp.concatenate([src, jnp.zeros((pad,), jnp.int32)])
    dsts = jnp.concatenate([dst, jnp.full((pad,), N, jnp.int32)])
    srcd = jnp.concatenate([src, jnp.full((pad,), N, jnp.int32)]).reshape(R, 128)
    zeros1 = jnp.zeros((NPAD,), f32)
    zeros2 = jnp.zeros((NPAD, H), f32)
    xp = jnp.concatenate([x, jnp.zeros((NPAD - N,), jnp.int32)]).reshape(NPAD, 1)
    embp = jnp.zeros((VOCAB, DP), f32).at[:, :D].set(emb.astype(f32))

    eye8 = jnp.eye(8, dtype=f32)

    def padw(w):
        wp = jnp.zeros((DP, DP), f32).at[:D, :D].set(w.astype(f32).T)
        return (jnp.kron(eye8, wp[:H]), jnp.kron(eye8, wp[H:]))

    def padb(b):
        bp = jnp.zeros((1, DP), f32).at[0, :D].set(b.astype(f32))
        return jnp.tile(bp, (1, 8))

    sel_a = jnp.kron(eye8, jnp.eye(DP, H, dtype=f32))
    sel_b = jnp.kron(eye8, jnp.eye(DP, H, -H, dtype=f32))
    wop = jnp.zeros((DP, 8), f32).at[:D, :C].set(Wout.astype(f32).T)
    wo = jnp.kron(eye8, wop)
    bo = jnp.tile(jnp.zeros((1, 8), f32).at[0, :C].set(bout.astype(f32)),
                  (1, 8))

    degp = _deg_call(srcd, zeros1)
    hnA, hnB, inv = _init_call(xp, embp, degp.reshape(NC, NPAD).T)
    invp16 = jnp.repeat(inv.reshape(NPAD), H).reshape(NPAD8, 128)
    for w, b in ((W0, b0), (W1, b1)):
        pA, pB = _agg_call(hnA, hnB, srcg, dsts, zeros2)
        wa, wb = padw(w)
        hnA8, hnB8 = _mid_call(pA.reshape(NC, NPAD8, 128),
                               pB.reshape(NC, NPAD8, 128),
                               invp16, wa, wb, padb(b), sel_a, sel_b)
        hnA = hnA8.reshape(NPAD, H)
        hnB = hnB8.reshape(NPAD, H)
    pA, pB = _agg_call(hnA, hnB, srcg, dsts, zeros2)
    wa, wb = padw(W2)
    out = _fin_call(pA.reshape(NC, NPAD8, 128), pB.reshape(NC, NPAD8, 128),
                    wa, wb, padb(b2), wo, bo)
    return out.reshape(NPAD, 8)[:N, :C]


# speedup vs baseline: 28.3729x; 1.3203x over previous
import functools

import jax
import jax.numpy as jnp
from jax import lax
from jax.experimental import pallas as pl
from jax.experimental.pallas import tpu as pltpu
from jax.experimental.pallas import tpu_sc as plsc

N = 100000
E = 1600000
VOCAB = 128
D = 30
C = 5
DP = 32
H = 16

NC, NS, L = 2, 16, 16
NW = NC * NS

NPAD = 100352
RSC = NPAD // NS

CH = 8
R = 12544
EPAD = R * 128
RT = R // NW
NIT = RT // CH

BLK = 2048
GRID = NPAD // BLK


def _sc_mesh():
    return plsc.VectorSubcoreMesh(
        core_axis_name="c", subcore_axis_name="s",
        num_cores=NC, num_subcores=NS)


def _deg_body(srcd, zeros1, degp, idx_v, ones_v, accum):
    cid = lax.axis_index("c")
    sid = lax.axis_index("s")
    wid = sid * NC + cid
    for i in range(128 // L):
        ones_v[pl.ds(i * L, L)] = jnp.ones((L,), jnp.float32)
    r0 = sid * RSC
    pltpu.sync_copy(zeros1.at[pl.ds(r0, RSC)], accum.at[pl.ds(r0, RSC)])
    plsc.subcore_barrier()
    base = wid * RT

    def it(t, carry):
        row = base + t * CH
        pltpu.sync_copy(srcd.at[pl.ds(row, CH)], idx_v)
        for j in range(CH):
            pltpu.sync_copy(ones_v, accum.at[idx_v.at[j]], add=True)
        return carry

    lax.fori_loop(0, NIT, it, 0)
    plsc.subcore_barrier()
    pltpu.sync_copy(accum.at[pl.ds(r0, RSC)],
                    degp.at[pl.ds(cid * NPAD + r0, RSC)])


_deg_call = pl.kernel(
    _deg_body,
    out_type=jax.ShapeDtypeStruct((NC * NPAD,), jnp.float32),
    mesh=_sc_mesh(),
    scratch_types=[
        pltpu.VMEM((CH, 128), jnp.int32),
        pltpu.VMEM((128,), jnp.float32),
        pltpu.VMEM_SHARED((NPAD,), jnp.float32),
    ],
    compiler_params=pltpu.CompilerParams(use_tc_tiling_on_sc=False),
)


CE = 512
NCHUNK = RT * 128 // CE
assert NCHUNK == 98


def _agg_body(hnA, hnB, srcg, dsts, zeros2, pA, pB,
              s0, d0, s1, d1, s2, d2, rows0, rows1, rows2, accum,
              semg0, semg1, semg2, sems0, sems1, sems2,
              semi0, semi1, semi2):
    cid = lax.axis_index("c")
    sid = lax.axis_index("s")
    wid = sid * NC + cid
    r0 = sid * RSC
    base = wid * RT * 128
    B = ((s0, d0, rows0, semg0, sems0, semi0),
         (s1, d1, rows1, semg1, sems1, semi1),
         (s2, d2, rows2, semg2, sems2, semi2))

    for half in range(2):
        hn = (hnA, hnB)[half]
        out = (pA, pB)[half]

        def idx(t, k):
            s, d, _, _, _, semi = B[k]
            pltpu.async_copy(srcg.at[pl.ds(base + t * CE, CE)], s, semi)
            pltpu.async_copy(dsts.at[pl.ds(base + t * CE, CE)], d, semi)

        def wait_i(k):
            s, d, _, _, _, semi = B[k]
            pltpu.make_async_copy(srcg.at[pl.ds(base, CE)], s, semi).wait()
            pltpu.make_async_copy(dsts.at[pl.ds(base, CE)], d, semi).wait()

        def gather(k):
            s, _, rows, semg, _, _ = B[k]
            pltpu.async_copy(hn.at[s], rows, semg)

        def wait_g(k):
            s, _, rows, semg, _, _ = B[k]
            pltpu.make_async_copy(hn.at[s], rows, semg).wait()

        def flush(k):
            _, d, rows, _, sems, _ = B[k]
            pltpu.async_copy(rows, accum.at[d], sems, add=True)

        def wait_s(k):
            _, d, rows, _, sems, _ = B[k]
            pltpu.make_async_copy(rows, accum.at[d], sems).wait()

        @pl.when(cid == 0)
        def _():
            pltpu.sync_copy(hn.at[pl.ds(r0, RSC)], accum.at[pl.ds(r0, RSC)])

        @pl.when(cid != 0)
        def _():
            pltpu.sync_copy(zeros2.at[pl.ds(r0, RSC)], accum.at[pl.ds(r0, RSC)])

        plsc.subcore_barrier()

        idx(0, 0)
        idx(1, 1)
        idx(2, 2)
        wait_i(0)
        gather(0)
        wait_i(1)
        gather(1)
        wait_g(0)
        flush(0)
        wait_s(0)
        idx(3, 0)
        wait_i(2)
        gather(2)
        wait_g(1)
        flush(1)

        def it_fixed(g, carry):
            for j in range(3):
                t = 3 * g + 2 + j
                kt = (2 + j) % 3
                kp = (kt + 2) % 3
                kn = (kt + 1) % 3
                wait_s(kp)
                idx(t + 2, kp)
                wait_i(kn)
                gather(kn)
                wait_g(kt)
                flush(kt)
            return carry

        lax.fori_loop(0, 31, it_fixed, 0)
        wait_s(1)
        idx(97, 1)
        wait_i(0)
        gather(0)
        wait_g(2)
        flush(2)
        wait_i(1)
        gather(1)
        wait_g(0)
        flush(0)
        wait_g(1)
        flush(1)
        wait_s(2)
        wait_s(0)
        wait_s(1)

        plsc.subcore_barrier()
        pltpu.sync_copy(accum.at[pl.ds(r0, RSC)], out.at[cid, pl.ds(r0, RSC)])
        if half == 0:
            plsc.subcore_barrier()


_agg_call = pl.kernel(
    _agg_body,
    out_type=(jax.ShapeDtypeStruct((NC, NPAD, H), jnp.float32),
              jax.ShapeDtypeStruct((NC, NPAD, H), jnp.float32)),
    mesh=_sc_mesh(),
    scratch_types=(
        [pltpu.VMEM((CE,), jnp.int32)] * 6
        + [pltpu.VMEM((CE, H), jnp.float32)] * 3
        + [pltpu.VMEM_SHARED((NPAD, H), jnp.float32)]
        + [pltpu.SemaphoreType.DMA] * 9
    ),
    compiler_params=pltpu.CompilerParams(use_tc_tiling_on_sc=False),
)


BLK8 = BLK // 8
NPAD8 = NPAD // 8


def _init_body(x_ref, emb_ref, d_ref, hnA_ref, hnB_ref, inv_ref):
    i = pl.program_id(0)
    xb = x_ref[...]
    onehot = (xb == lax.broadcasted_iota(jnp.int32, (BLK, VOCAB), 1)
              ).astype(jnp.float32)
    h0 = jnp.dot(onehot, emb_ref[...], preferred_element_type=jnp.float32)
    deg = d_ref[:, 0:1] + d_ref[:, 1:2] + 1.0
    row = i * BLK + lax.broadcasted_iota(jnp.int32, (BLK, 1), 0)
    inv = jnp.where(row < N, 1.0 / deg, 0.0)
    hn = h0 * inv
    hnA_ref[...] = hn[:, :H]
    hnB_ref[...] = hn[:, H:]
    inv_ref[...] = inv


_init_call = pl.pallas_call(
    _init_body,
    grid=(GRID,),
    in_specs=[
        pl.BlockSpec((BLK, 1), lambda i: (i, 0)),
        pl.BlockSpec((VOCAB, DP), lambda i: (0, 0)),
        pl.BlockSpec((BLK, NC), lambda i: (i, 0)),
    ],
    out_specs=[
        pl.BlockSpec((BLK, H), lambda i: (i, 0)),
        pl.BlockSpec((BLK, H), lambda i: (i, 0)),
        pl.BlockSpec((BLK, 1), lambda i: (i, 0)),
    ],
    out_shape=[
        jax.ShapeDtypeStruct((NPAD, H), jnp.float32),
        jax.ShapeDtypeStruct((NPAD, H), jnp.float32),
        jax.ShapeDtypeStruct((NPAD, 1), jnp.float32),
    ],
)


def _hidden_packed(pA_ref, pB_ref, wa_ref, wb_ref, b_ref):
    za = jnp.dot(pA_ref[0] + pA_ref[1], wa_ref[...],
                 preferred_element_type=jnp.float32)
    zb = jnp.dot(pB_ref[0] + pB_ref[1], wb_ref[...],
                 preferred_element_type=jnp.float32)
    return jnp.maximum(za + zb + b_ref[...], 0.0)


def _mid_body(pA_ref, pB_ref, inv_ref, wa_ref, wb_ref, b_ref, sa_ref, sb_ref,
              hnA_ref, hnB_ref):
    h = _hidden_packed(pA_ref, pB_ref, wa_ref, wb_ref, b_ref)
    ip = inv_ref[...]
    hnA_ref[...] = jnp.dot(h, sa_ref[...],
                           preferred_element_type=jnp.float32) * ip
    hnB_ref[...] = jnp.dot(h, sb_ref[...],
                           preferred_element_type=jnp.float32) * ip


_mid_call = pl.pallas_call(
    _mid_body,
    grid=(GRID,),
    in_specs=[
        pl.BlockSpec((NC, BLK8, 128), lambda i: (0, i, 0)),
        pl.BlockSpec((NC, BLK8, 128), lambda i: (0, i, 0)),
        pl.BlockSpec((BLK8, 128), lambda i: (i, 0)),
        pl.BlockSpec((128, 256), lambda i: (0, 0)),
        pl.BlockSpec((128, 256), lambda i: (0, 0)),
        pl.BlockSpec((1, 256), lambda i: (0, 0)),
        pl.BlockSpec((256, 128), lambda i: (0, 0)),
        pl.BlockSpec((256, 128), lambda i: (0, 0)),
    ],
    out_specs=[
        pl.BlockSpec((BLK8, 128), lambda i: (i, 0)),
        pl.BlockSpec((BLK8, 128), lambda i: (i, 0)),
    ],
    out_shape=[
        jax.ShapeDtypeStruct((NPAD8, 128), jnp.float32),
        jax.ShapeDtypeStruct((NPAD8, 128), jnp.float32),
    ],
)


def _fin_body(pA_ref, pB_ref, wa_ref, wb_ref, b_ref, wo_ref, bo_ref, out_ref):
    h = _hidden_packed(pA_ref, pB_ref, wa_ref, wb_ref, b_ref)
    out_ref[...] = (jnp.dot(h, wo_ref[...], preferred_element_type=jnp.float32)
                    + bo_ref[...])


_fin_call = pl.pallas_call(
    _fin_body,
    grid=(GRID,),
    in_specs=[
        pl.BlockSpec((NC, BLK8, 128), lambda i: (0, i, 0)),
        pl.BlockSpec((NC, BLK8, 128), lambda i: (0, i, 0)),
        pl.BlockSpec((128, 256), lambda i: (0, 0)),
        pl.BlockSpec((128, 256), lambda i: (0, 0)),
        pl.BlockSpec((1, 256), lambda i: (0, 0)),
        pl.BlockSpec((256, 64), lambda i: (0, 0)),
        pl.BlockSpec((1, 64), lambda i: (0, 0)),
    ],
    out_specs=pl.BlockSpec((BLK8, 64), lambda i: (i, 0)),
    out_shape=jax.ShapeDtypeStruct((NPAD8, 64), jnp.float32),
)


def kernel(x, edge_index, emb, W0, b0, W1, b1, W2, b2, Wout, bout):
    f32 = jnp.float32
    x = x.astype(jnp.int32)
    src = edge_index[0].astype(jnp.int32)
    dst = edge_index[1].astype(jnp.int32)
    pad = EPAD - E
    srcg = jnp.concatenate([src, jnp.zeros((pad,), jnp.int32)])
    dsts = jnp.concatenate([dst, jnp.full((pad,), N, jnp.int32)])
    srcd = jnp.concatenate([src, jnp.full((pad,), N, jnp.int32)]).reshape(R, 128)
    zeros1 = jnp.zeros((NPAD,), f32)
    zeros2 = jnp.zeros((NPAD, H), f32)
    xp = jnp.concatenate([x, jnp.zeros((NPAD - N,), jnp.int32)]).reshape(NPAD, 1)
    embp = jnp.zeros((VOCAB, DP), f32).at[:, :D].set(emb.astype(f32))

    eye8 = jnp.eye(8, dtype=f32)

    def padw(w):
        wp = jnp.zeros((DP, DP), f32).at[:D, :D].set(w.astype(f32).T)
        return (jnp.kron(eye8, wp[:H]), jnp.kron(eye8, wp[H:]))

    def padb(b):
        bp = jnp.zeros((1, DP), f32).at[0, :D].set(b.astype(f32))
        return jnp.tile(bp, (1, 8))

    sel_a = jnp.kron(eye8, jnp.eye(DP, H, dtype=f32))
    sel_b = jnp.kron(eye8, jnp.eye(DP, H, -H, dtype=f32))
    wop = jnp.zeros((DP, 8), f32).at[:D, :C].set(Wout.astype(f32).T)
    wo = jnp.kron(eye8, wop)
    bo = jnp.tile(jnp.zeros((1, 8), f32).at[0, :C].set(bout.astype(f32)),
                  (1, 8))

    degp = _deg_call(srcd, zeros1)
    hnA, hnB, inv = _init_call(xp, embp, degp.reshape(NC, NPAD).T)
    invp16 = jnp.repeat(inv.reshape(NPAD), H).reshape(NPAD8, 128)
    for w, b in ((W0, b0), (W1, b1)):
        pA, pB = _agg_call(hnA, hnB, srcg, dsts, zeros2)
        wa, wb = padw(w)
        hnA8, hnB8 = _mid_call(pA.reshape(NC, NPAD8, 128),
                               pB.reshape(NC, NPAD8, 128),
                               invp16, wa, wb, padb(b), sel_a, sel_b)
        hnA = hnA8.reshape(NPAD, H)
        hnB = hnB8.reshape(NPAD, H)
    pA, pB = _agg_call(hnA, hnB, srcg, dsts, zeros2)
    wa, wb = padw(W2)
    out = _fin_call(pA.reshape(NC, NPAD8, 128), pB.reshape(NC, NPAD8, 128),
                    wa, wb, padb(b2), wo, bo)
    return out.reshape(NPAD, 8)[:N, :C]
